# baseline (speedup 1.0000x reference)
import functools

import jax
import jax.numpy as jnp
from jax import lax
from jax.experimental import pallas as pl
from jax.experimental.pallas import tpu as pltpu
from jax.experimental.pallas import tpu_sc as plsc

NN = 10000
NP = 10240
EE = 160000
EPAD = 163840
NWORK = 32
EPT = EPAD // NWORK
EPT1 = EPAD // 16
CHUNK = 128
NCHUNK = EPT // CHUNK
NCHUNK1 = EPT1 // CHUNK
DD = 256
HH = 128
VV = 119
BLK = 512
GRID = NP // BLK

_HI = lax.Precision.HIGHEST
_mesh = plsc.VectorSubcoreMesh(core_axis_name="c", subcore_axis_name="s")
_mesh1 = plsc.VectorSubcoreMesh(core_axis_name="c", subcore_axis_name="s",
                                num_cores=1)


def _dot(a, b):
    return lax.dot_general(a, b, (((1,), (0,)), ((), ())),
                           precision=_HI, preferred_element_type=jnp.float32)


def _dot_t(a, b):
    return lax.dot_general(a, b, (((0,), (0,)), ((), ())),
                           precision=_HI, preferred_element_type=jnp.float32)


@functools.partial(
    pl.kernel, mesh=_mesh,
    out_type=jax.ShapeDtypeStruct((NWORK, NP), jnp.float32),
    compiler_params=pltpu.CompilerParams(needs_layout_passes=False),
    scratch_types=[pltpu.VMEM((NP,), jnp.float32),
                   pltpu.VMEM((CHUNK,), jnp.int32)])
def _sc_deg(dst_hbm, out_hbm, acc_v, idx_v):
    wid = lax.axis_index("s") * 2 + lax.axis_index("c")

    def zero(i, carry):
        acc_v[pl.ds(i * 16, 16)] = jnp.zeros((16,), jnp.float32)
        return carry
    lax.fori_loop(0, NP // 16, zero, 0)

    base = wid * EPT
    ones16 = jnp.ones((16,), jnp.float32)

    def chunk(ci, carry):
        pltpu.sync_copy(dst_hbm.at[pl.ds(base + ci * CHUNK, CHUNK)], idx_v)

        def grp(gi, c2):
            idx = idx_v[pl.ds(gi * 16, 16)]
            plsc.addupdate_scatter(acc_v, [idx], ones16)
            return c2
        return lax.fori_loop(0, CHUNK // 16, grp, carry)
    lax.fori_loop(0, NCHUNK, chunk, 0)
    pltpu.sync_copy(acc_v, out_hbm.at[wid])


@functools.partial(
    pl.kernel, mesh=_mesh,
    out_type=jax.ShapeDtypeStruct((NWORK, NP), jnp.float32),
    compiler_params=pltpu.CompilerParams(needs_layout_passes=False),
    scratch_types=[pltpu.VMEM((NP,), jnp.float32),
                   pltpu.VMEM((NP,), jnp.float32),
                   pltpu.VMEM((CHUNK,), jnp.int32),
                   pltpu.VMEM((CHUNK,), jnp.int32)])
def _sc_ctil(src_hbm, dst_hbm, dinv_hbm, out_hbm, acc_v, dv_v, sidx_v, didx_v):
    wid = lax.axis_index("s") * 2 + lax.axis_index("c")
    pltpu.sync_copy(dinv_hbm, dv_v)

    def zero(i, carry):
        acc_v[pl.ds(i * 16, 16)] = jnp.zeros((16,), jnp.float32)
        return carry
    lax.fori_loop(0, NP // 16, zero, 0)

    base = wid * EPT

    def chunk(ci, carry):
        pltpu.sync_copy(src_hbm.at[pl.ds(base + ci * CHUNK, CHUNK)], sidx_v)
        pltpu.sync_copy(dst_hbm.at[pl.ds(base + ci * CHUNK, CHUNK)], didx_v)

        def grp(gi, c2):
            sidx = sidx_v[pl.ds(gi * 16, 16)]
            didx = didx_v[pl.ds(gi * 16, 16)]
            val = plsc.load_gather(dv_v, [didx])
            plsc.addupdate_scatter(acc_v, [sidx], val)
            return c2
        return lax.fori_loop(0, CHUNK // 16, grp, carry)
    lax.fori_loop(0, NCHUNK, chunk, 0)
    pltpu.sync_copy(acc_v, out_hbm.at[wid])


@functools.partial(
    pl.kernel, mesh=_mesh1,
    out_type=jax.ShapeDtypeStruct((NP, HH), jnp.float32),
    compiler_params=pltpu.CompilerParams(needs_layout_passes=False),
    scratch_types=[pltpu.VMEM((CHUNK,), jnp.int32),
                   pltpu.VMEM((CHUNK,), jnp.int32),
                   pltpu.VMEM((CHUNK,), jnp.int32),
                   pltpu.VMEM((CHUNK,), jnp.int32),
                   pltpu.VMEM((CHUNK, HH), jnp.float32),
                   pltpu.VMEM((CHUNK, HH), jnp.float32),
                   pltpu.VMEM_SHARED((NP, HH), jnp.float32),
                   pltpu.SemaphoreType.DMA,
                   pltpu.SemaphoreType.DMA])
def _sc_rowscat(hs_hbm, src_hbm, dst_hbm, zeros_hbm, out_hbm,
                sidx0, sidx1, didx0, didx1, rows0, rows1, accs, sem0, sem1):
    s = lax.axis_index("s")
    wid = s
    rows_per = NP // 16
    r0 = s * rows_per
    pltpu.sync_copy(zeros_hbm.at[pl.ds(r0, rows_per)],
                    accs.at[pl.ds(r0, rows_per)])
    plsc.subcore_barrier()

    base = wid * EPT1
    bufs = ((sidx0, didx0, rows0, sem0), (sidx1, didx1, rows1, sem1))

    def load_fire(ci, b):
        sidx, didx, rows, sem = bufs[b]
        off = base + ci * CHUNK
        pltpu.sync_copy(src_hbm.at[pl.ds(off, CHUNK)], sidx)
        pltpu.sync_copy(dst_hbm.at[pl.ds(off, CHUNK)], didx)
        pltpu.async_copy(hs_hbm.at[sidx], rows, sem)

    load_fire(0, 0)

    def pair(p, carry):
        for b in range(2):
            ci = p * 2 + b
            sidx, didx, rows, sem = bufs[b]

            @pl.when(ci + 1 < NCHUNK1)
            def _():
                load_fire(ci + 1, 1 - b)

            pltpu.make_async_copy(hs_hbm.at[sidx], rows, sem).wait()
            pltpu.sync_copy(rows, accs.at[didx], add=True)
        return carry
    lax.fori_loop(0, NCHUNK1 // 2, pair, 0)

    plsc.subcore_barrier()
    pltpu.sync_copy(accs.at[pl.ds(r0, rows_per)],
                    out_hbm.at[pl.ds(r0, rows_per)])


def _t1_body(tok_ref, w1_ref, out_ref):
    out_ref[...] = _dot(tok_ref[...], w1_ref[...])


_tc_t1 = pl.pallas_call(
    _t1_body,
    out_shape=jax.ShapeDtypeStruct((128, HH), jnp.float32),
)


def _dinv_body(degp_ref, out_ref):
    g = pl.program_id(0)
    s = _dot_t(degp_ref[...], jnp.ones((NWORK, HH), jnp.float32))
    dv = lax.rsqrt(s + 1.0)
    row = lax.broadcasted_iota(jnp.int32, (BLK, HH), 0) + g * BLK
    out_ref[...] = jnp.where(row < NN, dv, 0.0)


_tc_dinv = pl.pallas_call(
    _dinv_body,
    grid=(GRID,),
    in_specs=[pl.BlockSpec((NWORK, BLK), lambda g: (0, g))],
    out_specs=pl.BlockSpec((BLK, HH), lambda g: (g, 0)),
    out_shape=jax.ShapeDtypeStruct((NP, HH), jnp.float32),
)


def _hs1_body(t8_ref, t1_ref, dv_ref, out_ref):
    t2d = _dot_t(t8_ref[...], jnp.ones((8, 128), jnp.float32))
    lane = lax.broadcasted_iota(jnp.int32, (BLK, 128), 1).astype(jnp.float32)
    oh = jnp.where(t2d == lane, 1.0, 0.0).astype(jnp.float32)
    out_ref[...] = _dot(oh, t1_ref[...]) * dv_ref[...]


_tc_hs1 = pl.pallas_call(
    _hs1_body,
    grid=(GRID,),
    in_specs=[pl.BlockSpec((8, BLK), lambda g: (0, g)),
              pl.BlockSpec((128, HH), lambda g: (0, 0)),
              pl.BlockSpec((BLK, HH), lambda g: (g, 0))],
    out_specs=pl.BlockSpec((BLK, HH), lambda g: (g, 0)),
    out_shape=jax.ShapeDtypeStruct((NP, HH), jnp.float32),
)


def _layer_body(p_ref, hs_ref, dv_ref, b_ref, w_ref, out_ref):
    dv = dv_ref[...]
    x = jnp.maximum((p_ref[...] + hs_ref[...]) * dv + b_ref[...], 0.0)
    out_ref[...] = _dot(x, w_ref[...]) * dv


_tc_layer = pl.pallas_call(
    _layer_body,
    grid=(GRID,),
    in_specs=[pl.BlockSpec((BLK, HH), lambda g: (g, 0)),
              pl.BlockSpec((BLK, HH), lambda g: (g, 0)),
              pl.BlockSpec((BLK, HH), lambda g: (g, 0)),
              pl.BlockSpec((1, HH), lambda g: (0, 0)),
              pl.BlockSpec((HH, HH), lambda g: (0, 0))],
    out_specs=pl.BlockSpec((BLK, HH), lambda g: (g, 0)),
    out_shape=jax.ShapeDtypeStruct((NP, HH), jnp.float32),
)


def _final_body(p_ref, hs_ref, dv_ref, b2_ref, cp_ref, w3_ref,
                b3_ref, r_ref, out_ref):
    g = pl.program_id(0)
    dv = dv_ref[...]
    x3 = jnp.maximum((p_ref[...] + hs_ref[...]) * dv + b2_ref[...], 0.0)
    c2d = _dot_t(cp_ref[...], jnp.ones((NWORK, HH), jnp.float32))
    w2d = dv * (c2d + dv)

    @pl.when(g == 0)
    def _():
        r_ref[...] = jnp.zeros((HH, HH), jnp.float32)

    r_ref[...] += _dot_t(w2d, x3)

    @pl.when(g == GRID - 1)
    def _():
        r = r_ref[0:1, :] * (1.0 / NN)
        out_ref[...] = _dot(r, w3_ref[...]) + b3_ref[...]


_tc_final = pl.pallas_call(
    _final_body,
    grid=(GRID,),
    in_specs=[pl.BlockSpec((BLK, HH), lambda g: (g, 0)),
              pl.BlockSpec((BLK, HH), lambda g: (g, 0)),
              pl.BlockSpec((BLK, HH), lambda g: (g, 0)),
              pl.BlockSpec((1, HH), lambda g: (0, 0)),
              pl.BlockSpec((NWORK, BLK), lambda g: (0, g)),
              pl.BlockSpec((HH, DD), lambda g: (0, 0)),
              pl.BlockSpec((1, DD), lambda g: (0, 0))],
    out_specs=[pl.BlockSpec((HH, HH), lambda g: (0, 0)),
               pl.BlockSpec((1, DD), lambda g: (0, 0))],
    out_shape=[jax.ShapeDtypeStruct((HH, HH), jnp.float32),
               jax.ShapeDtypeStruct((1, DD), jnp.float32)],
)


def kernel(atom_types, edge_index, tok_embed, W1, b1, W2, b2, W3, b3):
    f32 = jnp.float32
    src = edge_index[0].astype(jnp.int32)
    dst = edge_index[1].astype(jnp.int32)
    pad_e = EPAD - EE
    srcp = jnp.concatenate([src, jnp.zeros((pad_e,), jnp.int32)])
    dstp = jnp.concatenate([dst, jnp.full((pad_e,), NN, jnp.int32)])
    t8 = jnp.zeros((8, NP), f32).at[0, :NN].set(atom_types.astype(f32))
    tokp = jnp.zeros((128, DD), f32).at[:VV].set(tok_embed)
    zeros_big = jnp.zeros((NP, HH), f32)
    b1r = b1.reshape(1, HH)
    b2r = b2.reshape(1, HH)
    b3r = b3.reshape(1, DD)

    degp = _sc_deg(dstp)
    dinv2d = _tc_dinv(degp)
    dinv1d = dinv2d[:, 0]
    cp = _sc_ctil(srcp, dstp, dinv1d)
    t1 = _tc_t1(tokp, W1)
    hs1 = _tc_hs1(t8, t1, dinv2d)
    p1 = _sc_rowscat(hs1, srcp, dstp, zeros_big)
    hs2 = _tc_layer(p1, hs1, dinv2d, b1r, W2)
    p2 = _sc_rowscat(hs2, srcp, dstp, zeros_big)
    _, out = _tc_final(p2, hs2, dinv2d, b2r, cp, W3, b3r)
    return out[0]


# speedup vs baseline: 1.1245x; 1.1245x over previous
import functools

import jax
import jax.numpy as jnp
from jax import lax
from jax.experimental import pallas as pl
from jax.experimental.pallas import tpu as pltpu
from jax.experimental.pallas import tpu_sc as plsc

NN = 10000
NP = 10240
EE = 160000
EPAD = 163840
NWORK = 32
EPT = EPAD // NWORK
CHUNK = 128
NCHUNK = EPT // CHUNK
NBUF = 4
DD = 256
HH = 128
VV = 119
BLK = 512
GRID = NP // BLK

_HI = lax.Precision.HIGHEST
_mesh = plsc.VectorSubcoreMesh(core_axis_name="c", subcore_axis_name="s")


def _dot(a, b):
    return lax.dot_general(a, b, (((1,), (0,)), ((), ())),
                           precision=_HI, preferred_element_type=jnp.float32)


def _dot_t(a, b):
    return lax.dot_general(a, b, (((0,), (0,)), ((), ())),
                           precision=_HI, preferred_element_type=jnp.float32)


@functools.partial(
    pl.kernel, mesh=_mesh,
    out_type=jax.ShapeDtypeStruct((NWORK, NP), jnp.float32),
    compiler_params=pltpu.CompilerParams(needs_layout_passes=False),
    scratch_types=[pltpu.VMEM((NP,), jnp.float32),
                   pltpu.VMEM((CHUNK,), jnp.int32)])
def _sc_deg(dst_hbm, out_hbm, acc_v, idx_v):
    wid = lax.axis_index("s") * 2 + lax.axis_index("c")

    def zero(i, carry):
        acc_v[pl.ds(i * 16, 16)] = jnp.zeros((16,), jnp.float32)
        return carry
    lax.fori_loop(0, NP // 16, zero, 0)

    base = wid * EPT
    ones16 = jnp.ones((16,), jnp.float32)

    def chunk(ci, carry):
        pltpu.sync_copy(dst_hbm.at[pl.ds(base + ci * CHUNK, CHUNK)], idx_v)

        def grp(gi, c2):
            idx = idx_v[pl.ds(gi * 16, 16)]
            plsc.addupdate_scatter(acc_v, [idx], ones16)
            return c2
        return lax.fori_loop(0, CHUNK // 16, grp, carry)
    lax.fori_loop(0, NCHUNK, chunk, 0)
    pltpu.sync_copy(acc_v, out_hbm.at[wid])


@functools.partial(
    pl.kernel, mesh=_mesh,
    out_type=jax.ShapeDtypeStruct((NWORK, NP), jnp.float32),
    compiler_params=pltpu.CompilerParams(needs_layout_passes=False),
    scratch_types=[pltpu.VMEM((NP,), jnp.float32),
                   pltpu.VMEM((NP,), jnp.float32),
                   pltpu.VMEM((CHUNK,), jnp.int32),
                   pltpu.VMEM((CHUNK,), jnp.int32)])
def _sc_ctil(src_hbm, dst_hbm, dinv_hbm, out_hbm, acc_v, dv_v, sidx_v, didx_v):
    wid = lax.axis_index("s") * 2 + lax.axis_index("c")
    pltpu.sync_copy(dinv_hbm, dv_v)

    def zero(i, carry):
        acc_v[pl.ds(i * 16, 16)] = jnp.zeros((16,), jnp.float32)
        return carry
    lax.fori_loop(0, NP // 16, zero, 0)

    base = wid * EPT

    def chunk(ci, carry):
        pltpu.sync_copy(src_hbm.at[pl.ds(base + ci * CHUNK, CHUNK)], sidx_v)
        pltpu.sync_copy(dst_hbm.at[pl.ds(base + ci * CHUNK, CHUNK)], didx_v)

        def grp(gi, c2):
            sidx = sidx_v[pl.ds(gi * 16, 16)]
            didx = didx_v[pl.ds(gi * 16, 16)]
            val = plsc.load_gather(dv_v, [didx])
            plsc.addupdate_scatter(acc_v, [sidx], val)
            return c2
        return lax.fori_loop(0, CHUNK // 16, grp, carry)
    lax.fori_loop(0, NCHUNK, chunk, 0)
    pltpu.sync_copy(acc_v, out_hbm.at[wid])


@functools.partial(
    pl.kernel, mesh=_mesh,
    out_type=jax.ShapeDtypeStruct((2, NP, HH), jnp.float32),
    compiler_params=pltpu.CompilerParams(needs_layout_passes=False),
    scratch_types=[pltpu.VMEM((NCHUNK, CHUNK), jnp.int32),
                   pltpu.VMEM((NCHUNK, 1, CHUNK), jnp.int32),
                   pltpu.VMEM((CHUNK, HH), jnp.float32),
                   pltpu.VMEM((CHUNK, HH), jnp.float32),
                   pltpu.VMEM((CHUNK, HH), jnp.float32),
                   pltpu.VMEM((CHUNK, HH), jnp.float32),
                   pltpu.VMEM_SHARED((NP, HH), jnp.float32),
                   pltpu.SemaphoreType.DMA,
                   pltpu.SemaphoreType.DMA,
                   pltpu.SemaphoreType.DMA,
                   pltpu.SemaphoreType.DMA,
                   pltpu.SemaphoreType.DMA,
                   pltpu.SemaphoreType.DMA,
                   pltpu.SemaphoreType.DMA,
                   pltpu.SemaphoreType.DMA])
def _sc_rowscat(hs_hbm, src3_hbm, dst4_hbm, zeros_hbm, out_hbm,
                sidx_all, didx_all, rb0, rb1, rb2, rb3, accs,
                gs0, gs1, gs2, gs3, ss0, ss1, ss2, ss3):
    c = lax.axis_index("c")
    s = lax.axis_index("s")
    wid = s * 2 + c
    rows_per = NP // 16
    rr = s * rows_per
    pltpu.sync_copy(src3_hbm.at[wid], sidx_all)
    pltpu.sync_copy(dst4_hbm.at[wid], didx_all)
    pltpu.sync_copy(zeros_hbm.at[pl.ds(rr, rows_per)],
                    accs.at[pl.ds(rr, rows_per)])
    plsc.subcore_barrier()

    rows = (rb0, rb1, rb2, rb3)
    gsem = (gs0, gs1, gs2, gs3)

    def fire_gather(ci, b):
        pltpu.async_copy(hs_hbm.at[sidx_all.at[ci]], rows[b], gsem[b])

    fire_gather(0, 0)

    def pair(p, carry):
        for b in range(2):
            ci = p * 2 + b

            @pl.when(ci + 1 < NCHUNK)
            def _():
                fire_gather(ci + 1, 1 - b)

            pltpu.make_async_copy(
                hs_hbm.at[sidx_all.at[ci]], rows[b], gsem[b]).wait()
            pltpu.sync_copy(rows[b], accs.at[didx_all.at[ci, 0]], add=True)
        return carry
    lax.fori_loop(0, NCHUNK // 2, pair, 0)

    plsc.subcore_barrier()
    pltpu.sync_copy(accs.at[pl.ds(rr, rows_per)],
                    out_hbm.at[c, pl.ds(rr, rows_per)])


def _t1_body(tok_ref, w1_ref, out_ref):
    out_ref[...] = _dot(tok_ref[...], w1_ref[...])


_tc_t1 = pl.pallas_call(
    _t1_body,
    out_shape=jax.ShapeDtypeStruct((128, HH), jnp.float32),
)


def _dinv_body(degp_ref, out_ref):
    g = pl.program_id(0)
    s = _dot_t(degp_ref[...], jnp.ones((NWORK, HH), jnp.float32))
    dv = lax.rsqrt(s + 1.0)
    row = lax.broadcasted_iota(jnp.int32, (BLK, HH), 0) + g * BLK
    out_ref[...] = jnp.where(row < NN, dv, 0.0)


_tc_dinv = pl.pallas_call(
    _dinv_body,
    grid=(GRID,),
    in_specs=[pl.BlockSpec((NWORK, BLK), lambda g: (0, g))],
    out_specs=pl.BlockSpec((BLK, HH), lambda g: (g, 0)),
    out_shape=jax.ShapeDtypeStruct((NP, HH), jnp.float32),
)


def _hs1_body(t8_ref, t1_ref, dv_ref, out_ref):
    t2d = _dot_t(t8_ref[...], jnp.ones((8, 128), jnp.float32))
    lane = lax.broadcasted_iota(jnp.int32, (BLK, 128), 1).astype(jnp.float32)
    oh = jnp.where(t2d == lane, 1.0, 0.0).astype(jnp.float32)
    out_ref[...] = _dot(oh, t1_ref[...]) * dv_ref[...]


_tc_hs1 = pl.pallas_call(
    _hs1_body,
    grid=(GRID,),
    in_specs=[pl.BlockSpec((8, BLK), lambda g: (0, g)),
              pl.BlockSpec((128, HH), lambda g: (0, 0)),
              pl.BlockSpec((BLK, HH), lambda g: (g, 0))],
    out_specs=pl.BlockSpec((BLK, HH), lambda g: (g, 0)),
    out_shape=jax.ShapeDtypeStruct((NP, HH), jnp.float32),
)


def _layer_body(p0_ref, p1_ref, hs_ref, dv_ref, b_ref, w_ref, out_ref):
    dv = dv_ref[...]
    x = jnp.maximum((p0_ref[...] + p1_ref[...] + hs_ref[...]) * dv
                    + b_ref[...], 0.0)
    out_ref[...] = _dot(x, w_ref[...]) * dv


_tc_layer = pl.pallas_call(
    _layer_body,
    grid=(GRID,),
    in_specs=[pl.BlockSpec((BLK, HH), lambda g: (g, 0)),
              pl.BlockSpec((BLK, HH), lambda g: (g, 0)),
              pl.BlockSpec((BLK, HH), lambda g: (g, 0)),
              pl.BlockSpec((BLK, HH), lambda g: (g, 0)),
              pl.BlockSpec((1, HH), lambda g: (0, 0)),
              pl.BlockSpec((HH, HH), lambda g: (0, 0))],
    out_specs=pl.BlockSpec((BLK, HH), lambda g: (g, 0)),
    out_shape=jax.ShapeDtypeStruct((NP, HH), jnp.float32),
)


def _final_body(p0_ref, p1_ref, hs_ref, dv_ref, b2_ref, cp_ref, w3_ref,
                b3_ref, r_ref, out_ref):
    g = pl.program_id(0)
    dv = dv_ref[...]
    x3 = jnp.maximum((p0_ref[...] + p1_ref[...] + hs_ref[...]) * dv
                     + b2_ref[...], 0.0)
    c2d = _dot_t(cp_ref[...], jnp.ones((NWORK, HH), jnp.float32))
    w2d = dv * (c2d + dv)

    @pl.when(g == 0)
    def _():
        r_ref[...] = jnp.zeros((HH, HH), jnp.float32)

    r_ref[...] += _dot_t(w2d, x3)

    @pl.when(g == GRID - 1)
    def _():
        r = r_ref[0:1, :] * (1.0 / NN)
        out_ref[...] = _dot(r, w3_ref[...]) + b3_ref[...]


_tc_final = pl.pallas_call(
    _final_body,
    grid=(GRID,),
    in_specs=[pl.BlockSpec((BLK, HH), lambda g: (g, 0)),
              pl.BlockSpec((BLK, HH), lambda g: (g, 0)),
              pl.BlockSpec((BLK, HH), lambda g: (g, 0)),
              pl.BlockSpec((BLK, HH), lambda g: (g, 0)),
              pl.BlockSpec((1, HH), lambda g: (0, 0)),
              pl.BlockSpec((NWORK, BLK), lambda g: (0, g)),
              pl.BlockSpec((HH, DD), lambda g: (0, 0)),
              pl.BlockSpec((1, DD), lambda g: (0, 0))],
    out_specs=[pl.BlockSpec((HH, HH), lambda g: (0, 0)),
               pl.BlockSpec((1, DD), lambda g: (0, 0))],
    out_shape=[jax.ShapeDtypeStruct((HH, HH), jnp.float32),
               jax.ShapeDtypeStruct((1, DD), jnp.float32)],
)


def kernel(atom_types, edge_index, tok_embed, W1, b1, W2, b2, W3, b3):
    f32 = jnp.float32
    src = edge_index[0].astype(jnp.int32)
    dst = edge_index[1].astype(jnp.int32)
    pad_e = EPAD - EE
    srcp = jnp.concatenate([src, jnp.zeros((pad_e,), jnp.int32)])
    dstp = jnp.concatenate([dst, jnp.full((pad_e,), NN, jnp.int32)])
    src3 = srcp.reshape(NWORK, NCHUNK, CHUNK)
    dst4 = dstp.reshape(NWORK, NCHUNK, 1, CHUNK)
    t8 = jnp.zeros((8, NP), f32).at[0, :NN].set(atom_types.astype(f32))
    tokp = jnp.zeros((128, DD), f32).at[:VV].set(tok_embed)
    zeros_big = jnp.zeros((NP, HH), f32)
    b1r = b1.reshape(1, HH)
    b2r = b2.reshape(1, HH)
    b3r = b3.reshape(1, DD)

    degp = _sc_deg(dstp)
    dinv2d = _tc_dinv(degp)
    dinv1d = dinv2d[:, 0]
    cp = _sc_ctil(srcp, dstp, dinv1d)
    t1 = _tc_t1(tokp, W1)
    hs1 = _tc_hs1(t8, t1, dinv2d)
    p1 = _sc_rowscat(hs1, src3, dst4, zeros_big)
    hs2 = _tc_layer(p1[0], p1[1], hs1, dinv2d, b1r, W2)
    p2 = _sc_rowscat(hs2, src3, dst4, zeros_big)
    _, out = _tc_final(p2[0], p2[1], hs2, dinv2d, b2r, cp, W3, b3r)
    return out[0]


# speedup vs baseline: 1.5504x; 1.3787x over previous
import functools

import jax
import jax.numpy as jnp
from jax import lax
from jax.experimental import pallas as pl
from jax.experimental.pallas import tpu as pltpu
from jax.experimental.pallas import tpu_sc as plsc

NN = 10000
NP = 10240
EE = 160000
EPAD = 163840
NWORK = 32
EPT = EPAD // NWORK
CHUNK = 128
NCHUNK = EPT // CHUNK
NBUF = 4
DD = 256
HH = 128
VV = 119
BLK = 512
GRID = NP // BLK

_HI = lax.Precision.HIGHEST
_mesh = plsc.VectorSubcoreMesh(core_axis_name="c", subcore_axis_name="s")


def _dot(a, b):
    return lax.dot_general(a, b, (((1,), (0,)), ((), ())),
                           precision=_HI, preferred_element_type=jnp.float32)


def _dot_t(a, b):
    return lax.dot_general(a, b, (((0,), (0,)), ((), ())),
                           precision=_HI, preferred_element_type=jnp.float32)


@functools.partial(
    pl.kernel, mesh=_mesh,
    out_type=jax.ShapeDtypeStruct((NWORK, NP), jnp.float32),
    compiler_params=pltpu.CompilerParams(needs_layout_passes=False),
    scratch_types=[pltpu.VMEM((NP,), jnp.float32),
                   pltpu.VMEM((CHUNK,), jnp.int32)])
def _sc_deg(dst_hbm, out_hbm, acc_v, idx_v):
    wid = lax.axis_index("s") * 2 + lax.axis_index("c")

    def zero(i, carry):
        acc_v[pl.ds(i * 16, 16)] = jnp.zeros((16,), jnp.float32)
        return carry
    lax.fori_loop(0, NP // 16, zero, 0)

    base = wid * EPT
    ones16 = jnp.ones((16,), jnp.float32)

    def chunk(ci, carry):
        pltpu.sync_copy(dst_hbm.at[pl.ds(base + ci * CHUNK, CHUNK)], idx_v)

        def grp(gi, c2):
            idx = idx_v[pl.ds(gi * 16, 16)]
            plsc.addupdate_scatter(acc_v, [idx], ones16)
            return c2
        return lax.fori_loop(0, CHUNK // 16, grp, carry)
    lax.fori_loop(0, NCHUNK, chunk, 0)
    pltpu.sync_copy(acc_v, out_hbm.at[wid])


@functools.partial(
    pl.kernel, mesh=_mesh,
    out_type=[jax.ShapeDtypeStruct((2, NP * HH), jnp.float32),
              jax.ShapeDtypeStruct((NWORK, NP), jnp.float32)],
    compiler_params=pltpu.CompilerParams(needs_layout_passes=False),
    scratch_types=[pltpu.VMEM((NP,), jnp.float32),
                   pltpu.VMEM((NP,), jnp.int32),
                   pltpu.VMEM((NP,), jnp.float32),
                   pltpu.VMEM((CHUNK,), jnp.int32),
                   pltpu.VMEM((CHUNK,), jnp.int32),
                   pltpu.VMEM((CHUNK,), jnp.int32),
                   pltpu.VMEM((CHUNK,), jnp.float32),
                   pltpu.VMEM_SHARED((NP * HH,), jnp.float32)])
def _sc_hist(src_hbm, dst_hbm, dinv_hbm, types_hbm, zerosf_hbm,
             mout_hbm, cout_hbm,
             dv_v, ty_v, acc_v, sidx_v, didx_v, flat_v, vals_v, accm):
    c = lax.axis_index("c")
    s = lax.axis_index("s")
    wid = s * 2 + c
    rows_per_w = (NP * HH) // 16
    rr = s * rows_per_w
    pltpu.sync_copy(dinv_hbm, dv_v)
    pltpu.sync_copy(types_hbm, ty_v)
    pltpu.sync_copy(zerosf_hbm.at[pl.ds(rr, rows_per_w)],
                    accm.at[pl.ds(rr, rows_per_w)])

    def zero(i, carry):
        acc_v[pl.ds(i * 16, 16)] = jnp.zeros((16,), jnp.float32)
        return carry
    lax.fori_loop(0, NP // 16, zero, 0)
    plsc.subcore_barrier()

    base = wid * EPT

    def chunk(ci, carry):
        pltpu.sync_copy(src_hbm.at[pl.ds(base + ci * CHUNK, CHUNK)], sidx_v)
        pltpu.sync_copy(dst_hbm.at[pl.ds(base + ci * CHUNK, CHUNK)], didx_v)

        def grp(gi, c2):
            sidx = sidx_v[pl.ds(gi * 16, 16)]
            didx = didx_v[pl.ds(gi * 16, 16)]
            cval = plsc.load_gather(dv_v, [didx])
            plsc.addupdate_scatter(acc_v, [sidx], cval)
            mval = plsc.load_gather(dv_v, [sidx])
            t16 = plsc.load_gather(ty_v, [sidx])
            flat_v[pl.ds(gi * 16, 16)] = didx * HH + t16
            vals_v[pl.ds(gi * 16, 16)] = mval
            return c2
        lax.fori_loop(0, CHUNK // 16, grp, 0)
        pltpu.sync_copy(vals_v, accm.at[flat_v], add=True)
        return carry
    lax.fori_loop(0, NCHUNK, chunk, 0)
    pltpu.sync_copy(acc_v, cout_hbm.at[wid])
    plsc.subcore_barrier()
    pltpu.sync_copy(accm.at[pl.ds(rr, rows_per_w)],
                    mout_hbm.at[c, pl.ds(rr, rows_per_w)])


@functools.partial(
    pl.kernel, mesh=_mesh,
    out_type=jax.ShapeDtypeStruct((2, NP, HH), jnp.float32),
    compiler_params=pltpu.CompilerParams(needs_layout_passes=False),
    scratch_types=[pltpu.VMEM((NCHUNK, CHUNK), jnp.int32),
                   pltpu.VMEM((NCHUNK, 1, CHUNK), jnp.int32),
                   pltpu.VMEM((CHUNK, HH), jnp.float32),
                   pltpu.VMEM((CHUNK, HH), jnp.float32),
                   pltpu.VMEM((CHUNK, HH), jnp.float32),
                   pltpu.VMEM((CHUNK, HH), jnp.float32),
                   pltpu.VMEM_SHARED((NP, HH), jnp.float32),
                   pltpu.SemaphoreType.DMA,
                   pltpu.SemaphoreType.DMA,
                   pltpu.SemaphoreType.DMA,
                   pltpu.SemaphoreType.DMA,
                   pltpu.SemaphoreType.DMA,
                   pltpu.SemaphoreType.DMA,
                   pltpu.SemaphoreType.DMA,
                   pltpu.SemaphoreType.DMA])
def _sc_rowscat(hs_hbm, src3_hbm, dst4_hbm, zeros_hbm, out_hbm,
                sidx_all, didx_all, rb0, rb1, rb2, rb3, accs,
                gs0, gs1, gs2, gs3, ss0, ss1, ss2, ss3):
    c = lax.axis_index("c")
    s = lax.axis_index("s")
    wid = s * 2 + c
    rows_per = NP // 16
    rr = s * rows_per
    pltpu.sync_copy(src3_hbm.at[wid], sidx_all)
    pltpu.sync_copy(dst4_hbm.at[wid], didx_all)
    pltpu.sync_copy(zeros_hbm.at[pl.ds(rr, rows_per)],
                    accs.at[pl.ds(rr, rows_per)])
    plsc.subcore_barrier()

    rows = (rb0, rb1, rb2, rb3)
    gsem = (gs0, gs1, gs2, gs3)

    def fire_gather(ci, b):
        pltpu.async_copy(hs_hbm.at[sidx_all.at[ci]], rows[b], gsem[b])

    fire_gather(0, 0)

    def pair(p, carry):
        for b in range(2):
            ci = p * 2 + b

            @pl.when(ci + 1 < NCHUNK)
            def _():
                fire_gather(ci + 1, 1 - b)

            pltpu.make_async_copy(
                hs_hbm.at[sidx_all.at[ci]], rows[b], gsem[b]).wait()
            pltpu.sync_copy(rows[b], accs.at[didx_all.at[ci, 0]], add=True)
        return carry
    lax.fori_loop(0, NCHUNK // 2, pair, 0)

    plsc.subcore_barrier()
    pltpu.sync_copy(accs.at[pl.ds(rr, rows_per)],
                    out_hbm.at[c, pl.ds(rr, rows_per)])


def _t1_body(tok_ref, w1_ref, out_ref):
    out_ref[...] = _dot(tok_ref[...], w1_ref[...])


_tc_t1 = pl.pallas_call(
    _t1_body,
    out_shape=jax.ShapeDtypeStruct((128, HH), jnp.float32),
)


def _dinv_body(degp_ref, out_ref):
    g = pl.program_id(0)
    s = _dot_t(degp_ref[...], jnp.ones((NWORK, HH), jnp.float32))
    dv = lax.rsqrt(s + 1.0)
    row = lax.broadcasted_iota(jnp.int32, (BLK, HH), 0) + g * BLK
    out_ref[...] = jnp.where(row < NN, dv, 0.0)


_tc_dinv = pl.pallas_call(
    _dinv_body,
    grid=(GRID,),
    in_specs=[pl.BlockSpec((NWORK, BLK), lambda g: (0, g))],
    out_specs=pl.BlockSpec((BLK, HH), lambda g: (g, 0)),
    out_shape=jax.ShapeDtypeStruct((NP, HH), jnp.float32),
)


def _hs1_body(t8_ref, t1_ref, dv_ref, out_ref):
    t2d = _dot_t(t8_ref[...], jnp.ones((8, 128), jnp.float32))
    lane = lax.broadcasted_iota(jnp.int32, (BLK, 128), 1).astype(jnp.float32)
    oh = jnp.where(t2d == lane, 1.0, 0.0).astype(jnp.float32)
    out_ref[...] = _dot(oh, t1_ref[...]) * dv_ref[...]


_tc_hs1 = pl.pallas_call(
    _hs1_body,
    grid=(GRID,),
    in_specs=[pl.BlockSpec((8, BLK), lambda g: (0, g)),
              pl.BlockSpec((128, HH), lambda g: (0, 0)),
              pl.BlockSpec((BLK, HH), lambda g: (g, 0))],
    out_specs=pl.BlockSpec((BLK, HH), lambda g: (g, 0)),
    out_shape=jax.ShapeDtypeStruct((NP, HH), jnp.float32),
)


def _layer1_body(m0_ref, m1_ref, t1_ref, hs_ref, dv_ref, b_ref, w_ref,
                 out_ref):
    dv = dv_ref[...]
    s1 = _dot(m0_ref[...] + m1_ref[...], t1_ref[...])
    x = jnp.maximum((s1 + hs_ref[...]) * dv + b_ref[...], 0.0)
    out_ref[...] = _dot(x, w_ref[...]) * dv


_tc_layer1 = pl.pallas_call(
    _layer1_body,
    grid=(GRID,),
    in_specs=[pl.BlockSpec((BLK, HH), lambda g: (g, 0)),
              pl.BlockSpec((BLK, HH), lambda g: (g, 0)),
              pl.BlockSpec((128, HH), lambda g: (0, 0)),
              pl.BlockSpec((BLK, HH), lambda g: (g, 0)),
              pl.BlockSpec((BLK, HH), lambda g: (g, 0)),
              pl.BlockSpec((1, HH), lambda g: (0, 0)),
              pl.BlockSpec((HH, HH), lambda g: (0, 0))],
    out_specs=pl.BlockSpec((BLK, HH), lambda g: (g, 0)),
    out_shape=jax.ShapeDtypeStruct((NP, HH), jnp.float32),
)


def _layer_body(p0_ref, p1_ref, hs_ref, dv_ref, b_ref, w_ref, out_ref):
    dv = dv_ref[...]
    x = jnp.maximum((p0_ref[...] + p1_ref[...] + hs_ref[...]) * dv
                    + b_ref[...], 0.0)
    out_ref[...] = _dot(x, w_ref[...]) * dv


_tc_layer = pl.pallas_call(
    _layer_body,
    grid=(GRID,),
    in_specs=[pl.BlockSpec((BLK, HH), lambda g: (g, 0)),
              pl.BlockSpec((BLK, HH), lambda g: (g, 0)),
              pl.BlockSpec((BLK, HH), lambda g: (g, 0)),
              pl.BlockSpec((BLK, HH), lambda g: (g, 0)),
              pl.BlockSpec((1, HH), lambda g: (0, 0)),
              pl.BlockSpec((HH, HH), lambda g: (0, 0))],
    out_specs=pl.BlockSpec((BLK, HH), lambda g: (g, 0)),
    out_shape=jax.ShapeDtypeStruct((NP, HH), jnp.float32),
)


def _final_body(p0_ref, p1_ref, hs_ref, dv_ref, b2_ref, cp_ref, w3_ref,
                b3_ref, r_ref, out_ref):
    g = pl.program_id(0)
    dv = dv_ref[...]
    x3 = jnp.maximum((p0_ref[...] + p1_ref[...] + hs_ref[...]) * dv
                     + b2_ref[...], 0.0)
    c2d = _dot_t(cp_ref[...], jnp.ones((NWORK, HH), jnp.float32))
    w2d = dv * (c2d + dv)

    @pl.when(g == 0)
    def _():
        r_ref[...] = jnp.zeros((HH, HH), jnp.float32)

    r_ref[...] += _dot_t(w2d, x3)

    @pl.when(g == GRID - 1)
    def _():
        r = r_ref[0:1, :] * (1.0 / NN)
        out_ref[...] = _dot(r, w3_ref[...]) + b3_ref[...]


_tc_final = pl.pallas_call(
    _final_body,
    grid=(GRID,),
    in_specs=[pl.BlockSpec((BLK, HH), lambda g: (g, 0)),
              pl.BlockSpec((BLK, HH), lambda g: (g, 0)),
              pl.BlockSpec((BLK, HH), lambda g: (g, 0)),
              pl.BlockSpec((BLK, HH), lambda g: (g, 0)),
              pl.BlockSpec((1, HH), lambda g: (0, 0)),
              pl.BlockSpec((NWORK, BLK), lambda g: (0, g)),
              pl.BlockSpec((HH, DD), lambda g: (0, 0)),
              pl.BlockSpec((1, DD), lambda g: (0, 0))],
    out_specs=[pl.BlockSpec((HH, HH), lambda g: (0, 0)),
               pl.BlockSpec((1, DD), lambda g: (0, 0))],
    out_shape=[jax.ShapeDtypeStruct((HH, HH), jnp.float32),
               jax.ShapeDtypeStruct((1, DD), jnp.float32)],
)


def kernel(atom_types, edge_index, tok_embed, W1, b1, W2, b2, W3, b3):
    f32 = jnp.float32
    src = edge_index[0].astype(jnp.int32)
    dst = edge_index[1].astype(jnp.int32)
    pad_e = EPAD - EE
    srcp = jnp.concatenate([src, jnp.zeros((pad_e,), jnp.int32)])
    dstp = jnp.concatenate([dst, jnp.full((pad_e,), NN, jnp.int32)])
    src3 = srcp.reshape(NWORK, NCHUNK, CHUNK)
    dst4 = dstp.reshape(NWORK, NCHUNK, 1, CHUNK)
    t8 = jnp.zeros((8, NP), f32).at[0, :NN].set(atom_types.astype(f32))
    tokp = jnp.zeros((128, DD), f32).at[:VV].set(tok_embed)
    zeros_big = jnp.zeros((NP, HH), f32)
    b1r = b1.reshape(1, HH)
    b2r = b2.reshape(1, HH)
    b3r = b3.reshape(1, DD)

    types_p = jnp.zeros((NP,), jnp.int32).at[:NN].set(atom_types.astype(jnp.int32))
    zeros_flat = zeros_big.reshape(NP * HH)

    degp = _sc_deg(dstp)
    dinv2d = _tc_dinv(degp)
    dinv1d = dinv2d[:, 0]
    mp, cp = _sc_hist(srcp, dstp, dinv1d, types_p, zeros_flat)
    m0 = mp[0].reshape(NP, HH)
    m1 = mp[1].reshape(NP, HH)
    t1 = _tc_t1(tokp, W1)
    hs1 = _tc_hs1(t8, t1, dinv2d)
    hs2 = _tc_layer1(m0, m1, t1, hs1, dinv2d, b1r, W2)
    p2 = _sc_rowscat(hs2, src3, dst4, zeros_big)
    _, out = _tc_final(p2[0], p2[1], hs2, dinv2d, b2r, cp, W3, b3r)
    return out[0]


# speedup vs baseline: 1.5518x; 1.0009x over previous
import functools

import jax
import jax.numpy as jnp
from jax import lax
from jax.experimental import pallas as pl
from jax.experimental.pallas import tpu as pltpu
from jax.experimental.pallas import tpu_sc as plsc

NN = 10000
NP = 10240
EE = 160000
EPAD = 163840
NWORK = 32
EPT = EPAD // NWORK
CHUNK = 128
NCHUNK = EPT // CHUNK
NBUF = 3
NPA = 10112
DD = 256
HH = 128
VV = 119
BLK = 512
GRID = NP // BLK

_HI = lax.Precision.HIGHEST
_mesh = plsc.VectorSubcoreMesh(core_axis_name="c", subcore_axis_name="s")


def _dot(a, b):
    return lax.dot_general(a, b, (((1,), (0,)), ((), ())),
                           precision=_HI, preferred_element_type=jnp.float32)


def _dot_t(a, b):
    return lax.dot_general(a, b, (((0,), (0,)), ((), ())),
                           precision=_HI, preferred_element_type=jnp.float32)


@functools.partial(
    pl.kernel, mesh=_mesh,
    out_type=jax.ShapeDtypeStruct((NWORK, NP), jnp.float32),
    compiler_params=pltpu.CompilerParams(needs_layout_passes=False),
    scratch_types=[pltpu.VMEM((NP,), jnp.float32),
                   pltpu.VMEM((CHUNK,), jnp.int32)])
def _sc_deg(dst_hbm, out_hbm, acc_v, idx_v):
    wid = lax.axis_index("s") * 2 + lax.axis_index("c")

    def zero(i, carry):
        acc_v[pl.ds(i * 16, 16)] = jnp.zeros((16,), jnp.float32)
        return carry
    lax.fori_loop(0, NP // 16, zero, 0)

    base = wid * EPT
    ones16 = jnp.ones((16,), jnp.float32)

    def chunk(ci, carry):
        pltpu.sync_copy(dst_hbm.at[pl.ds(base + ci * CHUNK, CHUNK)], idx_v)

        def grp(gi, c2):
            idx = idx_v[pl.ds(gi * 16, 16)]
            plsc.addupdate_scatter(acc_v, [idx], ones16)
            return c2
        return lax.fori_loop(0, CHUNK // 16, grp, carry)
    lax.fori_loop(0, NCHUNK, chunk, 0)
    pltpu.sync_copy(acc_v, out_hbm.at[wid])


@functools.partial(
    pl.kernel, mesh=_mesh,
    out_type=[jax.ShapeDtypeStruct((2, NP * HH), jnp.float32),
              jax.ShapeDtypeStruct((NWORK, NP), jnp.float32)],
    compiler_params=pltpu.CompilerParams(needs_layout_passes=False),
    scratch_types=[pltpu.VMEM((NP,), jnp.float32),
                   pltpu.VMEM((NP,), jnp.int32),
                   pltpu.VMEM((NP,), jnp.float32),
                   pltpu.VMEM((CHUNK,), jnp.int32),
                   pltpu.VMEM((CHUNK,), jnp.int32),
                   pltpu.VMEM((CHUNK,), jnp.int32),
                   pltpu.VMEM((CHUNK,), jnp.float32),
                   pltpu.VMEM_SHARED((NP * HH,), jnp.float32)])
def _sc_hist(src_hbm, dst_hbm, dinv_hbm, types_hbm, zerosf_hbm,
             mout_hbm, cout_hbm,
             dv_v, ty_v, acc_v, sidx_v, didx_v, flat_v, vals_v, accm):
    c = lax.axis_index("c")
    s = lax.axis_index("s")
    wid = s * 2 + c
    rows_per_w = (NP * HH) // 16
    rr = s * rows_per_w
    pltpu.sync_copy(dinv_hbm, dv_v)
    pltpu.sync_copy(types_hbm, ty_v)
    pltpu.sync_copy(zerosf_hbm.at[pl.ds(rr, rows_per_w)],
                    accm.at[pl.ds(rr, rows_per_w)])

    def zero(i, carry):
        acc_v[pl.ds(i * 16, 16)] = jnp.zeros((16,), jnp.float32)
        return carry
    lax.fori_loop(0, NP // 16, zero, 0)
    plsc.subcore_barrier()

    base = wid * EPT

    def chunk(ci, carry):
        pltpu.sync_copy(src_hbm.at[pl.ds(base + ci * CHUNK, CHUNK)], sidx_v)
        pltpu.sync_copy(dst_hbm.at[pl.ds(base + ci * CHUNK, CHUNK)], didx_v)

        def grp(gi, c2):
            sidx = sidx_v[pl.ds(gi * 16, 16)]
            didx = didx_v[pl.ds(gi * 16, 16)]
            cval = plsc.load_gather(dv_v, [didx])
            plsc.addupdate_scatter(acc_v, [sidx], cval)
            mval = plsc.load_gather(dv_v, [sidx])
            t16 = plsc.load_gather(ty_v, [sidx])
            flat_v[pl.ds(gi * 16, 16)] = didx * HH + t16
            vals_v[pl.ds(gi * 16, 16)] = mval
            return c2
        lax.fori_loop(0, CHUNK // 16, grp, 0)
        pltpu.sync_copy(vals_v, accm.at[flat_v], add=True)
        return carry
    lax.fori_loop(0, NCHUNK, chunk, 0)
    pltpu.sync_copy(acc_v, cout_hbm.at[wid])
    plsc.subcore_barrier()
    pltpu.sync_copy(accm.at[pl.ds(rr, rows_per_w)],
                    mout_hbm.at[c, pl.ds(rr, rows_per_w)])


@functools.partial(
    pl.kernel, mesh=_mesh,
    out_type=jax.ShapeDtypeStruct((2, NP, HH), jnp.float32),
    compiler_params=pltpu.CompilerParams(needs_layout_passes=False),
    scratch_types=[pltpu.VMEM((CHUNK,), jnp.int32),
                   pltpu.VMEM((CHUNK,), jnp.int32),
                   pltpu.VMEM((CHUNK,), jnp.int32),
                   pltpu.VMEM((CHUNK,), jnp.int32),
                   pltpu.VMEM((CHUNK,), jnp.int32),
                   pltpu.VMEM((CHUNK,), jnp.int32),
                   pltpu.VMEM((CHUNK, HH), jnp.float32),
                   pltpu.VMEM((CHUNK, HH), jnp.float32),
                   pltpu.VMEM((CHUNK, HH), jnp.float32),
                   pltpu.VMEM_SHARED((NPA, HH), jnp.float32),
                   pltpu.SemaphoreType.DMA,
                   pltpu.SemaphoreType.DMA,
                   pltpu.SemaphoreType.DMA])
def _sc_rowscat(hs_hbm, src_hbm, dst_hbm, zeros_hbm, out_hbm,
                si0, si1, si2, di0, di1, di2, rb0, rb1, rb2, accs,
                gs0, gs1, gs2):
    c = lax.axis_index("c")
    s = lax.axis_index("s")
    wid = s * 2 + c
    rows_per = NPA // 16
    rr = s * rows_per
    pltpu.sync_copy(zeros_hbm.at[pl.ds(rr, rows_per)],
                    accs.at[pl.ds(rr, rows_per)])
    plsc.subcore_barrier()

    base = wid * EPT
    sidx = (si0, si1, si2)
    didx = (di0, di1, di2)
    rows = (rb0, rb1, rb2)
    gsem = (gs0, gs1, gs2)

    def load_fire(ci, b):
        off = base + ci * CHUNK
        pltpu.sync_copy(src_hbm.at[pl.ds(off, CHUNK)], sidx[b])
        pltpu.sync_copy(dst_hbm.at[pl.ds(off, CHUNK)], didx[b])
        pltpu.async_copy(hs_hbm.at[sidx[b]], rows[b], gsem[b])

    load_fire(0, 0)
    load_fire(1, 1)

    def tri(q, carry):
        for b in range(NBUF):
            ci = q * NBUF + b

            @pl.when(ci + 2 < NCHUNK)
            def _():
                load_fire(ci + 2, (b + 2) % NBUF)

            pltpu.make_async_copy(
                hs_hbm.at[sidx[b]], rows[b], gsem[b]).wait()
            pltpu.sync_copy(rows[b], accs.at[didx[b]], add=True)
        return carry
    lax.fori_loop(0, (NCHUNK - 1) // NBUF, tri, 0)

    lb = (NCHUNK - 1) % NBUF
    pltpu.make_async_copy(
        hs_hbm.at[sidx[lb]], rows[lb], gsem[lb]).wait()
    pltpu.sync_copy(rows[lb], accs.at[didx[lb]], add=True)

    plsc.subcore_barrier()
    pltpu.sync_copy(accs.at[pl.ds(rr, rows_per)],
                    out_hbm.at[c, pl.ds(rr, rows_per)])


def _t1_body(tok_ref, w1_ref, out_ref):
    out_ref[...] = _dot(tok_ref[...], w1_ref[...])


_tc_t1 = pl.pallas_call(
    _t1_body,
    out_shape=jax.ShapeDtypeStruct((128, HH), jnp.float32),
)


def _dinv_body(degp_ref, out_ref):
    g = pl.program_id(0)
    s = _dot_t(degp_ref[...], jnp.ones((NWORK, HH), jnp.float32))
    dv = lax.rsqrt(s + 1.0)
    row = lax.broadcasted_iota(jnp.int32, (BLK, HH), 0) + g * BLK
    out_ref[...] = jnp.where(row < NN, dv, 0.0)


_tc_dinv = pl.pallas_call(
    _dinv_body,
    grid=(GRID,),
    in_specs=[pl.BlockSpec((NWORK, BLK), lambda g: (0, g))],
    out_specs=pl.BlockSpec((BLK, HH), lambda g: (g, 0)),
    out_shape=jax.ShapeDtypeStruct((NP, HH), jnp.float32),
)


def _hs1_body(t8_ref, t1_ref, dv_ref, out_ref):
    t2d = _dot_t(t8_ref[...], jnp.ones((8, 128), jnp.float32))
    lane = lax.broadcasted_iota(jnp.int32, (BLK, 128), 1).astype(jnp.float32)
    oh = jnp.where(t2d == lane, 1.0, 0.0).astype(jnp.float32)
    out_ref[...] = _dot(oh, t1_ref[...]) * dv_ref[...]


_tc_hs1 = pl.pallas_call(
    _hs1_body,
    grid=(GRID,),
    in_specs=[pl.BlockSpec((8, BLK), lambda g: (0, g)),
              pl.BlockSpec((128, HH), lambda g: (0, 0)),
              pl.BlockSpec((BLK, HH), lambda g: (g, 0))],
    out_specs=pl.BlockSpec((BLK, HH), lambda g: (g, 0)),
    out_shape=jax.ShapeDtypeStruct((NP, HH), jnp.float32),
)


def _layer1_body(m0_ref, m1_ref, t1_ref, hs_ref, dv_ref, b_ref, w_ref,
                 out_ref):
    dv = dv_ref[...]
    s1 = _dot(m0_ref[...] + m1_ref[...], t1_ref[...])
    x = jnp.maximum((s1 + hs_ref[...]) * dv + b_ref[...], 0.0)
    out_ref[...] = _dot(x, w_ref[...]) * dv


_tc_layer1 = pl.pallas_call(
    _layer1_body,
    grid=(GRID,),
    in_specs=[pl.BlockSpec((BLK, HH), lambda g: (g, 0)),
              pl.BlockSpec((BLK, HH), lambda g: (g, 0)),
              pl.BlockSpec((128, HH), lambda g: (0, 0)),
              pl.BlockSpec((BLK, HH), lambda g: (g, 0)),
              pl.BlockSpec((BLK, HH), lambda g: (g, 0)),
              pl.BlockSpec((1, HH), lambda g: (0, 0)),
              pl.BlockSpec((HH, HH), lambda g: (0, 0))],
    out_specs=pl.BlockSpec((BLK, HH), lambda g: (g, 0)),
    out_shape=jax.ShapeDtypeStruct((NP, HH), jnp.float32),
)


def _layer_body(p0_ref, p1_ref, hs_ref, dv_ref, b_ref, w_ref, out_ref):
    dv = dv_ref[...]
    x = jnp.maximum((p0_ref[...] + p1_ref[...] + hs_ref[...]) * dv
                    + b_ref[...], 0.0)
    out_ref[...] = _dot(x, w_ref[...]) * dv


_tc_layer = pl.pallas_call(
    _layer_body,
    grid=(GRID,),
    in_specs=[pl.BlockSpec((BLK, HH), lambda g: (g, 0)),
              pl.BlockSpec((BLK, HH), lambda g: (g, 0)),
              pl.BlockSpec((BLK, HH), lambda g: (g, 0)),
              pl.BlockSpec((BLK, HH), lambda g: (g, 0)),
              pl.BlockSpec((1, HH), lambda g: (0, 0)),
              pl.BlockSpec((HH, HH), lambda g: (0, 0))],
    out_specs=pl.BlockSpec((BLK, HH), lambda g: (g, 0)),
    out_shape=jax.ShapeDtypeStruct((NP, HH), jnp.float32),
)


def _final_body(p0_ref, p1_ref, hs_ref, dv_ref, b2_ref, cp_ref, w3_ref,
                b3_ref, r_ref, out_ref):
    g = pl.program_id(0)
    dv = dv_ref[...]
    p = jnp.where(dv > 0.0, p0_ref[...] + p1_ref[...], 0.0)
    x3 = jnp.maximum((p + hs_ref[...]) * dv + b2_ref[...], 0.0)
    c2d = _dot_t(cp_ref[...], jnp.ones((NWORK, HH), jnp.float32))
    w2d = dv * (c2d + dv)

    @pl.when(g == 0)
    def _():
        r_ref[...] = jnp.zeros((HH, HH), jnp.float32)

    r_ref[...] += _dot_t(w2d, x3)

    @pl.when(g == GRID - 1)
    def _():
        r = r_ref[0:1, :] * (1.0 / NN)
        out_ref[...] = _dot(r, w3_ref[...]) + b3_ref[...]


_tc_final = pl.pallas_call(
    _final_body,
    grid=(GRID,),
    in_specs=[pl.BlockSpec((BLK, HH), lambda g: (g, 0)),
              pl.BlockSpec((BLK, HH), lambda g: (g, 0)),
              pl.BlockSpec((BLK, HH), lambda g: (g, 0)),
              pl.BlockSpec((BLK, HH), lambda g: (g, 0)),
              pl.BlockSpec((1, HH), lambda g: (0, 0)),
              pl.BlockSpec((NWORK, BLK), lambda g: (0, g)),
              pl.BlockSpec((HH, DD), lambda g: (0, 0)),
              pl.BlockSpec((1, DD), lambda g: (0, 0))],
    out_specs=[pl.BlockSpec((HH, HH), lambda g: (0, 0)),
               pl.BlockSpec((1, DD), lambda g: (0, 0))],
    out_shape=[jax.ShapeDtypeStruct((HH, HH), jnp.float32),
               jax.ShapeDtypeStruct((1, DD), jnp.float32)],
)


def kernel(atom_types, edge_index, tok_embed, W1, b1, W2, b2, W3, b3):
    f32 = jnp.float32
    src = edge_index[0].astype(jnp.int32)
    dst = edge_index[1].astype(jnp.int32)
    pad_e = EPAD - EE
    srcp = jnp.concatenate([src, jnp.zeros((pad_e,), jnp.int32)])
    dstp = jnp.concatenate([dst, jnp.full((pad_e,), NN, jnp.int32)])
    t8 = jnp.zeros((8, NP), f32).at[0, :NN].set(atom_types.astype(f32))
    tokp = jnp.zeros((128, DD), f32).at[:VV].set(tok_embed)
    zeros_big = jnp.zeros((NP, HH), f32)
    b1r = b1.reshape(1, HH)
    b2r = b2.reshape(1, HH)
    b3r = b3.reshape(1, DD)

    types_p = jnp.zeros((NP,), jnp.int32).at[:NN].set(atom_types.astype(jnp.int32))
    zeros_flat = zeros_big.reshape(NP * HH)

    degp = _sc_deg(dstp)
    dinv2d = _tc_dinv(degp)
    dinv1d = dinv2d[:, 0]
    mp, cp = _sc_hist(srcp, dstp, dinv1d, types_p, zeros_flat)
    m0 = mp[0].reshape(NP, HH)
    m1 = mp[1].reshape(NP, HH)
    t1 = _tc_t1(tokp, W1)
    hs1 = _tc_hs1(t8, t1, dinv2d)
    hs2 = _tc_layer1(m0, m1, t1, hs1, dinv2d, b1r, W2)
    p2 = _sc_rowscat(hs2, srcp, dstp, zeros_big)
    _, out = _tc_final(p2[0], p2[1], hs2, dinv2d, b2r, cp, W3, b3r)
    return out[0]


# speedup vs baseline: 1.5523x; 1.0003x over previous
import functools

import jax
import jax.numpy as jnp
from jax import lax
from jax.experimental import pallas as pl
from jax.experimental.pallas import tpu as pltpu
from jax.experimental.pallas import tpu_sc as plsc

NN = 10000
NP = 10240
EE = 160000
EPAD = 163840
NWORK = 32
EPT = EPAD // NWORK
CHUNK = 128
NCHUNK = EPT // CHUNK
NBUF = 3
NPA = 10112
DD = 256
HH = 128
VV = 119
BLK = 512
GRID = NP // BLK

_HI = lax.Precision.HIGHEST
_mesh = plsc.VectorSubcoreMesh(core_axis_name="c", subcore_axis_name="s")


def _dot(a, b):
    return lax.dot_general(a, b, (((1,), (0,)), ((), ())),
                           precision=_HI, preferred_element_type=jnp.float32)


def _dot_t(a, b):
    return lax.dot_general(a, b, (((0,), (0,)), ((), ())),
                           precision=_HI, preferred_element_type=jnp.float32)


@functools.partial(
    pl.kernel, mesh=_mesh,
    out_type=jax.ShapeDtypeStruct((NWORK, NP), jnp.float32),
    compiler_params=pltpu.CompilerParams(needs_layout_passes=False),
    scratch_types=[pltpu.VMEM((NP,), jnp.float32),
                   pltpu.VMEM((CHUNK,), jnp.int32)])
def _sc_deg(dst_hbm, out_hbm, acc_v, idx_v):
    wid = lax.axis_index("s") * 2 + lax.axis_index("c")

    def zero(i, carry):
        acc_v[pl.ds(i * 16, 16)] = jnp.zeros((16,), jnp.float32)
        return carry
    lax.fori_loop(0, NP // 16, zero, 0)

    base = wid * EPT
    ones16 = jnp.ones((16,), jnp.float32)

    def chunk(ci, carry):
        pltpu.sync_copy(dst_hbm.at[pl.ds(base + ci * CHUNK, CHUNK)], idx_v)

        def grp(gi, c2):
            idx = idx_v[pl.ds(gi * 16, 16)]
            plsc.addupdate_scatter(acc_v, [idx], ones16)
            return c2
        return lax.fori_loop(0, CHUNK // 16, grp, carry)
    lax.fori_loop(0, NCHUNK, chunk, 0)
    pltpu.sync_copy(acc_v, out_hbm.at[wid])


@functools.partial(
    pl.kernel, mesh=_mesh,
    out_type=[jax.ShapeDtypeStruct((2, NP * HH), jnp.float32),
              jax.ShapeDtypeStruct((NWORK, NP), jnp.float32)],
    compiler_params=pltpu.CompilerParams(needs_layout_passes=False),
    scratch_types=[pltpu.VMEM((NP,), jnp.float32),
                   pltpu.VMEM((NP,), jnp.int32),
                   pltpu.VMEM((NP,), jnp.float32),
                   pltpu.VMEM((CHUNK,), jnp.int32),
                   pltpu.VMEM((CHUNK,), jnp.int32),
                   pltpu.VMEM((CHUNK,), jnp.int32),
                   pltpu.VMEM((CHUNK,), jnp.float32),
                   pltpu.VMEM_SHARED((NP * HH,), jnp.float32)])
def _sc_hist(src_hbm, dst_hbm, dinv_hbm, types_hbm, zerosf_hbm,
             mout_hbm, cout_hbm,
             dv_v, ty_v, acc_v, sidx_v, didx_v, flat_v, vals_v, accm):
    c = lax.axis_index("c")
    s = lax.axis_index("s")
    wid = s * 2 + c
    rows_per_w = (NP * HH) // 16
    rr = s * rows_per_w
    pltpu.sync_copy(dinv_hbm, dv_v)
    pltpu.sync_copy(types_hbm, ty_v)
    pltpu.sync_copy(zerosf_hbm.at[pl.ds(rr, rows_per_w)],
                    accm.at[pl.ds(rr, rows_per_w)])

    def zero(i, carry):
        acc_v[pl.ds(i * 16, 16)] = jnp.zeros((16,), jnp.float32)
        return carry
    lax.fori_loop(0, NP // 16, zero, 0)
    plsc.subcore_barrier()

    base = wid * EPT

    def chunk(ci, carry):
        pltpu.sync_copy(src_hbm.at[pl.ds(base + ci * CHUNK, CHUNK)], sidx_v)
        pltpu.sync_copy(dst_hbm.at[pl.ds(base + ci * CHUNK, CHUNK)], didx_v)

        def grp(gi, c2):
            sidx = sidx_v[pl.ds(gi * 16, 16)]
            didx = didx_v[pl.ds(gi * 16, 16)]
            cval = plsc.load_gather(dv_v, [didx])
            plsc.addupdate_scatter(acc_v, [sidx], cval)
            mval = plsc.load_gather(dv_v, [sidx])
            t16 = plsc.load_gather(ty_v, [sidx])
            flat_v[pl.ds(gi * 16, 16)] = didx * HH + t16
            vals_v[pl.ds(gi * 16, 16)] = mval
            return c2
        lax.fori_loop(0, CHUNK // 16, grp, 0)
        pltpu.sync_copy(vals_v, accm.at[flat_v], add=True)
        return carry
    lax.fori_loop(0, NCHUNK, chunk, 0)
    pltpu.sync_copy(acc_v, cout_hbm.at[wid])
    plsc.subcore_barrier()
    pltpu.sync_copy(accm.at[pl.ds(rr, rows_per_w)],
                    mout_hbm.at[c, pl.ds(rr, rows_per_w)])


@functools.partial(
    pl.kernel, mesh=_mesh,
    out_type=jax.ShapeDtypeStruct((2, NP, HH), jnp.float32),
    compiler_params=pltpu.CompilerParams(needs_layout_passes=False),
    scratch_types=[pltpu.VMEM((CHUNK,), jnp.int32),
                   pltpu.VMEM((CHUNK,), jnp.int32),
                   pltpu.VMEM((CHUNK,), jnp.int32),
                   pltpu.VMEM((CHUNK,), jnp.int32),
                   pltpu.VMEM((CHUNK,), jnp.int32),
                   pltpu.VMEM((CHUNK,), jnp.int32),
                   pltpu.VMEM((CHUNK, HH), jnp.float32),
                   pltpu.VMEM((CHUNK, HH), jnp.float32),
                   pltpu.VMEM((CHUNK, HH), jnp.float32),
                   pltpu.VMEM_SHARED((NPA, HH), jnp.float32),
                   pltpu.SemaphoreType.DMA,
                   pltpu.SemaphoreType.DMA,
                   pltpu.SemaphoreType.DMA])
def _sc_rowscat(hs_hbm, src_hbm, dst_hbm, zeros_hbm, out_hbm,
                si0, si1, si2, di0, di1, di2, rb0, rb1, rb2, accs,
                gs0, gs1, gs2):
    c = lax.axis_index("c")
    s = lax.axis_index("s")
    wid = s * 2 + c
    rows_per = NPA // 16
    rr = s * rows_per
    pltpu.sync_copy(zeros_hbm.at[pl.ds(rr, rows_per)],
                    accs.at[pl.ds(rr, rows_per)])
    plsc.subcore_barrier()

    base = wid * EPT
    sidx = (si0, si1, si2)
    didx = (di0, di1, di2)
    rows = (rb0, rb1, rb2)
    gsem = (gs0, gs1, gs2)

    def load_fire(ci, b):
        off = base + ci * CHUNK
        pltpu.sync_copy(src_hbm.at[pl.ds(off, CHUNK)], sidx[b])
        pltpu.sync_copy(dst_hbm.at[pl.ds(off, CHUNK)], didx[b])
        pltpu.async_copy(hs_hbm.at[sidx[b]], rows[b], gsem[b])

    load_fire(0, 0)
    load_fire(1, 1)

    def tri(q, carry):
        for b in range(NBUF):
            ci = q * NBUF + b

            @pl.when(ci + 2 < NCHUNK)
            def _():
                load_fire(ci + 2, (b + 2) % NBUF)

            pltpu.make_async_copy(
                hs_hbm.at[sidx[b]], rows[b], gsem[b]).wait()
            pltpu.sync_copy(rows[b], accs.at[didx[b]], add=True)
        return carry
    lax.fori_loop(0, (NCHUNK - 1) // NBUF, tri, 0)

    lb = (NCHUNK - 1) % NBUF
    pltpu.make_async_copy(
        hs_hbm.at[sidx[lb]], rows[lb], gsem[lb]).wait()
    pltpu.sync_copy(rows[lb], accs.at[didx[lb]], add=True)

    plsc.subcore_barrier()
    pltpu.sync_copy(accs.at[pl.ds(rr, rows_per)],
                    out_hbm.at[c, pl.ds(rr, rows_per)])


def _t1_body(tok_ref, w1_ref, out_ref):
    out_ref[...] = _dot(tok_ref[...], w1_ref[...])


_tc_t1 = pl.pallas_call(
    _t1_body,
    out_shape=jax.ShapeDtypeStruct((128, HH), jnp.float32),
)


def _dinv_body(degp_ref, out_ref):
    g = pl.program_id(0)
    s = _dot_t(degp_ref[...], jnp.ones((NWORK, HH), jnp.float32))
    dv = lax.rsqrt(s + 1.0)
    row = lax.broadcasted_iota(jnp.int32, (BLK, HH), 0) + g * BLK
    out_ref[...] = jnp.where(row < NN, dv, 0.0)


_tc_dinv = pl.pallas_call(
    _dinv_body,
    grid=(GRID,),
    in_specs=[pl.BlockSpec((NWORK, BLK), lambda g: (0, g))],
    out_specs=pl.BlockSpec((BLK, HH), lambda g: (g, 0)),
    out_shape=jax.ShapeDtypeStruct((NP, HH), jnp.float32),
)


def _hs1_body(t8_ref, t1_ref, dv_ref, out_ref):
    t2d = _dot_t(t8_ref[...], jnp.ones((8, 128), jnp.float32))
    lane = lax.broadcasted_iota(jnp.int32, (BLK, 128), 1).astype(jnp.float32)
    oh = jnp.where(t2d == lane, 1.0, 0.0).astype(jnp.float32)
    out_ref[...] = _dot(oh, t1_ref[...]) * dv_ref[...]


_tc_hs1 = pl.pallas_call(
    _hs1_body,
    grid=(GRID,),
    in_specs=[pl.BlockSpec((8, BLK), lambda g: (0, g)),
              pl.BlockSpec((128, HH), lambda g: (0, 0)),
              pl.BlockSpec((BLK, HH), lambda g: (g, 0))],
    out_specs=pl.BlockSpec((BLK, HH), lambda g: (g, 0)),
    out_shape=jax.ShapeDtypeStruct((NP, HH), jnp.float32),
)


def _layer1_body(m0_ref, m1_ref, t1_ref, hs_ref, dv_ref, b_ref, w_ref,
                 out_ref):
    dv = dv_ref[...]
    s1 = _dot(m0_ref[...] + m1_ref[...], t1_ref[...])
    x = jnp.maximum((s1 + hs_ref[...]) * dv + b_ref[...], 0.0)
    out_ref[...] = _dot(x, w_ref[...]) * dv


_tc_layer1 = pl.pallas_call(
    _layer1_body,
    grid=(GRID,),
    in_specs=[pl.BlockSpec((BLK, HH), lambda g: (g, 0)),
              pl.BlockSpec((BLK, HH), lambda g: (g, 0)),
              pl.BlockSpec((128, HH), lambda g: (0, 0)),
              pl.BlockSpec((BLK, HH), lambda g: (g, 0)),
              pl.BlockSpec((BLK, HH), lambda g: (g, 0)),
              pl.BlockSpec((1, HH), lambda g: (0, 0)),
              pl.BlockSpec((HH, HH), lambda g: (0, 0))],
    out_specs=pl.BlockSpec((BLK, HH), lambda g: (g, 0)),
    out_shape=jax.ShapeDtypeStruct((NP, HH), jnp.float32),
)


def _final_body(p0_ref, p1_ref, hs_ref, dv_ref, b2_ref, cp_ref, w3_ref,
                b3_ref, r_ref, out_ref):
    g = pl.program_id(0)
    dv = dv_ref[...]
    p = jnp.where(dv > 0.0, p0_ref[...] + p1_ref[...], 0.0)
    x3 = jnp.maximum((p + hs_ref[...]) * dv + b2_ref[...], 0.0)
    c2d = _dot_t(cp_ref[...], jnp.ones((NWORK, HH), jnp.float32))
    w2d = dv * (c2d + dv)

    @pl.when(g == 0)
    def _():
        r_ref[...] = jnp.zeros((HH, HH), jnp.float32)

    r_ref[...] += _dot_t(w2d, x3)

    @pl.when(g == GRID - 1)
    def _():
        r = r_ref[0:1, :] * (1.0 / NN)
        out_ref[...] = _dot(r, w3_ref[...]) + b3_ref[...]


_tc_final = pl.pallas_call(
    _final_body,
    grid=(GRID,),
    in_specs=[pl.BlockSpec((BLK, HH), lambda g: (g, 0)),
              pl.BlockSpec((BLK, HH), lambda g: (g, 0)),
              pl.BlockSpec((BLK, HH), lambda g: (g, 0)),
              pl.BlockSpec((BLK, HH), lambda g: (g, 0)),
              pl.BlockSpec((1, HH), lambda g: (0, 0)),
              pl.BlockSpec((NWORK, BLK), lambda g: (0, g)),
              pl.BlockSpec((HH, DD), lambda g: (0, 0)),
              pl.BlockSpec((1, DD), lambda g: (0, 0))],
    out_specs=[pl.BlockSpec((HH, HH), lambda g: (0, 0)),
               pl.BlockSpec((1, DD), lambda g: (0, 0))],
    out_shape=[jax.ShapeDtypeStruct((HH, HH), jnp.float32),
               jax.ShapeDtypeStruct((1, DD), jnp.float32)],
)


def kernel(atom_types, edge_index, tok_embed, W1, b1, W2, b2, W3, b3):
    f32 = jnp.float32
    src = edge_index[0].astype(jnp.int32)
    dst = edge_index[1].astype(jnp.int32)
    pad_e = EPAD - EE
    srcp = jnp.concatenate([src, jnp.zeros((pad_e,), jnp.int32)])
    dstp = jnp.concatenate([dst, jnp.full((pad_e,), NN, jnp.int32)])
    t8 = jnp.zeros((8, NP), f32).at[0, :NN].set(atom_types.astype(f32))
    tokp = jnp.zeros((128, DD), f32).at[:VV].set(tok_embed)
    zeros_big = jnp.zeros((NP, HH), f32)
    b1r = b1.reshape(1, HH)
    b2r = b2.reshape(1, HH)
    b3r = b3.reshape(1, DD)

    types_p = jnp.zeros((NP,), jnp.int32).at[:NN].set(atom_types.astype(jnp.int32))
    zeros_flat = zeros_big.reshape(NP * HH)

    degp = _sc_deg(dstp)
    dinv2d = _tc_dinv(degp)
    dinv1d = dinv2d[:, 0]
    mp, cp = _sc_hist(srcp, dstp, dinv1d, types_p, zeros_flat)
    m0 = mp[0].reshape(NP, HH)
    m1 = mp[1].reshape(NP, HH)
    t1 = _tc_t1(tokp, W1)
    hs1 = _tc_hs1(t8, t1, dinv2d)
    hs2 = _tc_layer1(m0, m1, t1, hs1, dinv2d, b1r, W2)
    p2 = _sc_rowscat(hs2, srcp, dstp, zeros_big)
    _, out = _tc_final(p2[0], p2[1], hs2, dinv2d, b2r, cp, W3, b3r)
    return out[0]
